# Initial kernel scaffold; baseline (speedup 1.0000x reference)
import jax
import jax.numpy as jnp
from jax.experimental import pallas as pl


def kernel(wft_ics, bft_ics, stm, ft_weight, ft_bias, fc_w, fc_b):
    raise NotImplementedError("write your pallas kernel here")


# speedup vs baseline: 23.7604x; 23.7604x over previous
import functools

import jax
import jax.numpy as jnp
from jax import lax
from jax.experimental import pallas as pl
from jax.experimental.pallas import tpu as pltpu
from jax.experimental.pallas import tpu_sc as plsc

N_FEATURES = 40960
D = 257
DH = 256
BATCH = 16384
K = 32
NBAGS = 2 * BATCH
NC, NS = 2, 16
NW = NC * NS
BAGS_PER_W = NBAGS // NW
BCH = 128
NCH = BAGS_PER_W // BCH
GROWS = 128
GBAGS = GROWS // K
NG = BCH // GBAGS
L = 16


def _mesh():
    return plsc.VectorSubcoreMesh(
        core_axis_name="c", subcore_axis_name="s", num_cores=NC, num_subcores=NS
    )


@functools.cache
def _make_embed_bag():
    return functools.partial(
        pl.kernel,
        out_type=jax.ShapeDtypeStruct((NBAGS, DH), jnp.float32),
        mesh=_mesh(),
        compiler_params=pltpu.CompilerParams(needs_layout_passes=False),
        scratch_types=[
            pltpu.VMEM((BCH * K,), jnp.int32),
            pltpu.VMEM((2, GROWS, DH), jnp.float32),
            pltpu.VMEM((BCH, DH), jnp.float32),
            pltpu.SemaphoreType.DMA,
            pltpu.SemaphoreType.DMA,
        ],
    )(_embed_bag_body)


def _embed_bag_body(icsf_hbm, table_hbm, out_hbm, idxf, rows, acc, sem0, sem1):
    wid = lax.axis_index("s") * NC + lax.axis_index("c")
    sems = (sem0, sem1)

    def fire(g, buf):
        pltpu.async_copy(
            table_hbm.at[idxf.at[pl.ds(g * GROWS, GROWS)]],
            rows.at[buf], sems[buf])

    def wait(buf):
        pltpu.make_async_copy(
            table_hbm.at[idxf.at[pl.ds(0, GROWS)]], rows.at[buf],
            sems[buf]).wait()

    def reduce_buf(g, buf):
        rb = rows.at[buf]

        def bag_body(bag, c1):
            def col_body(cb, c2):
                s = pl.ds(cb * L, L)
                v = rb[bag * K, s]
                for r in range(1, K):
                    v = v + rb[bag * K + r, s]
                acc[g * GBAGS + bag, s] = v
                return c2

            lax.fori_loop(0, DH // L, col_body, 0)
            return c1

        lax.fori_loop(0, GBAGS, bag_body, 0)

    def chunk_body(ci, carry):
        base = wid * BAGS_PER_W + ci * BCH
        pltpu.sync_copy(icsf_hbm.at[pl.ds(base * K, BCH * K)], idxf)
        fire(0, 0)

        def pipe_body(g2, c1):
            g = 2 * g2
            fire(g + 1, 1)
            wait(0)
            reduce_buf(g, 0)

            @pl.when(g2 != NG // 2 - 1)
            def _():
                fire(g + 2, 0)

            wait(1)
            reduce_buf(g + 1, 1)
            return c1

        lax.fori_loop(0, NG // 2, pipe_body, 0)
        pltpu.sync_copy(acc, out_hbm.at[pl.ds(base, BCH)])
        return carry

    lax.fori_loop(0, NCH, chunk_body, 0)


@functools.cache
def _make_psqt():
    return functools.partial(
        pl.kernel,
        out_type=jax.ShapeDtypeStruct((NBAGS,), jnp.float32),
        mesh=_mesh(),
        compiler_params=pltpu.CompilerParams(needs_layout_passes=False),
        scratch_types=[
            pltpu.VMEM((K, BCH), jnp.int32),
            pltpu.VMEM((N_FEATURES,), jnp.float32),
            pltpu.VMEM((BCH,), jnp.float32),
        ],
    )(_psqt_body)


def _psqt_body(icst_hbm, psqt_hbm, outp_hbm, idxt, psqt_v, pacc):
    wid = lax.axis_index("s") * NC + lax.axis_index("c")
    pltpu.sync_copy(psqt_hbm, psqt_v)

    def chunk_body(ci, carry):
        base = wid * BAGS_PER_W + ci * BCH
        pltpu.sync_copy(icst_hbm.at[:, pl.ds(base, BCH)], idxt)
        for i in range(BCH // L):
            pacc[pl.ds(i * L, L)] = jnp.zeros((L,), jnp.float32)

        def psum(j, c1):
            for i in range(BCH // L):
                s = pl.ds(i * L, L)
                pacc[s] += plsc.load_gather(psqt_v, [idxt[j, s]])
            return c1

        lax.fori_loop(0, K, psum, 0)
        pltpu.sync_copy(pacc, outp_hbm.at[pl.ds(base, BCH)])
        return carry

    lax.fori_loop(0, NCH, chunk_body, 0)


def _fc_body(w_ref, b_ref, wp_ref, bp_ref, stm_ref, bias_ref, fcw_ref,
             fcb_ref, out_ref):
    bias = bias_ref[...]
    wfts = w_ref[...] + bias
    bfts = b_ref[...] + bias
    s = stm_ref[...]
    x1 = (1.0 - s) * wfts + s * bfts
    x2 = (1.0 - s) * bfts + s * wfts
    fcw = fcw_ref[...]
    fca, fcbb = fcw[:, :DH], fcw[:, DH:]
    acc = jnp.sum(jnp.clip(x1, 0.0, 1.0) * fca, axis=1, keepdims=True)
    acc = acc + jnp.sum(jnp.clip(x2, 0.0, 1.0) * fcbb, axis=1, keepdims=True)
    out_ref[...] = acc + fcb_ref[...] + (wp_ref[...] - bp_ref[...]) * (0.5 - s)


def kernel(wft_ics, bft_ics, stm, ft_weight, ft_bias, fc_w, fc_b):
    ics = jnp.concatenate([wft_ics, bft_ics], axis=0)
    ics_flat = ics.reshape(-1)
    ics_t = ics.T
    table = ft_weight[:, :DH]
    psqt_col = ft_weight[:, DH]
    acc = _make_embed_bag()(ics_flat, table)
    psqt = _make_psqt()(ics_t, psqt_col)
    psqt2 = psqt.reshape(NBAGS, 1)

    bm = 512
    nb = BATCH // bm
    out = pl.pallas_call(
        _fc_body,
        grid=(nb,),
        in_specs=[
            pl.BlockSpec((bm, DH), lambda i: (i, 0)),
            pl.BlockSpec((bm, DH), lambda i: (i + nb, 0)),
            pl.BlockSpec((bm, 1), lambda i: (i, 0)),
            pl.BlockSpec((bm, 1), lambda i: (i + nb, 0)),
            pl.BlockSpec((bm, 1), lambda i: (i, 0)),
            pl.BlockSpec((1, DH), lambda i: (0, 0)),
            pl.BlockSpec((1, 512), lambda i: (0, 0)),
            pl.BlockSpec((1, 1), lambda i: (0, 0)),
        ],
        out_specs=pl.BlockSpec((bm, 1), lambda i: (i, 0)),
        out_shape=jax.ShapeDtypeStruct((BATCH, 1), jnp.float32),
    )(acc, acc, psqt2, psqt2, stm, ft_bias[:DH].reshape(1, DH), fc_w,
      fc_b.reshape(1, 1))
    return out


# speedup vs baseline: 27.2763x; 1.1480x over previous
import functools

import jax
import jax.numpy as jnp
from jax import lax
from jax.experimental import pallas as pl
from jax.experimental.pallas import tpu as pltpu
from jax.experimental.pallas import tpu_sc as plsc

N_FEATURES = 40960
D = 257
DH = 256
BATCH = 16384
K = 32
NBAGS = 2 * BATCH
NC, NS = 2, 16
NW = NC * NS
BAGS_PER_W = NBAGS // NW
BCH = 128
NCH = BAGS_PER_W // BCH
GROWS = 128
GBAGS = GROWS // K
NG = BCH // GBAGS
L = 16


def _mesh():
    return plsc.VectorSubcoreMesh(
        core_axis_name="c", subcore_axis_name="s", num_cores=NC, num_subcores=NS
    )


@functools.cache
def _make_embed_bag():
    return functools.partial(
        pl.kernel,
        out_type=jax.ShapeDtypeStruct((NBAGS, DH), jnp.float32),
        mesh=_mesh(),
        compiler_params=pltpu.CompilerParams(needs_layout_passes=False),
        scratch_types=[
            pltpu.VMEM((BCH * K,), jnp.int32),
            pltpu.VMEM((2, GROWS, DH), jnp.float32),
            pltpu.VMEM((BCH, DH), jnp.float32),
            pltpu.SemaphoreType.DMA,
            pltpu.SemaphoreType.DMA,
        ],
    )(_embed_bag_body)


def _embed_bag_body(icsf_hbm, table_hbm, out_hbm, idxf, rows, acc, sem0, sem1):
    wid = lax.axis_index("s") * NC + lax.axis_index("c")
    sems = (sem0, sem1)

    def fire(g, buf):
        pltpu.async_copy(
            table_hbm.at[idxf.at[pl.ds(g * GROWS, GROWS)]],
            rows.at[buf], sems[buf])

    def wait(buf):
        pltpu.make_async_copy(
            table_hbm.at[idxf.at[pl.ds(0, GROWS)]], rows.at[buf],
            sems[buf]).wait()

    def reduce_buf(g, buf):
        rb = rows.at[buf]

        def bag_body(bag, c1):
            r0 = bag * K
            for cb in range(DH // L):
                s = pl.ds(cb * L, L)
                a0 = rb[r0, s] + rb[r0 + 4, s]
                a1 = rb[r0 + 1, s] + rb[r0 + 5, s]
                a2 = rb[r0 + 2, s] + rb[r0 + 6, s]
                a3 = rb[r0 + 3, s] + rb[r0 + 7, s]
                for r in range(8, K, 4):
                    a0 = a0 + rb[r0 + r, s]
                    a1 = a1 + rb[r0 + r + 1, s]
                    a2 = a2 + rb[r0 + r + 2, s]
                    a3 = a3 + rb[r0 + r + 3, s]
                acc[g * GBAGS + bag, s] = (a0 + a1) + (a2 + a3)
            return c1

        lax.fori_loop(0, GBAGS, bag_body, 0)

    def chunk_body(ci, carry):
        base = wid * BAGS_PER_W + ci * BCH
        pltpu.sync_copy(icsf_hbm.at[pl.ds(base * K, BCH * K)], idxf)
        fire(0, 0)

        def pipe_body(g2, c1):
            g = 2 * g2
            fire(g + 1, 1)
            wait(0)
            reduce_buf(g, 0)

            @pl.when(g2 != NG // 2 - 1)
            def _():
                fire(g + 2, 0)

            wait(1)
            reduce_buf(g + 1, 1)
            return c1

        lax.fori_loop(0, NG // 2, pipe_body, 0)
        pltpu.sync_copy(acc, out_hbm.at[pl.ds(base, BCH)])
        return carry

    lax.fori_loop(0, NCH, chunk_body, 0)


@functools.cache
def _make_psqt():
    return functools.partial(
        pl.kernel,
        out_type=jax.ShapeDtypeStruct((NBAGS,), jnp.float32),
        mesh=_mesh(),
        compiler_params=pltpu.CompilerParams(needs_layout_passes=False),
        scratch_types=[
            pltpu.VMEM((K, BCH), jnp.int32),
            pltpu.VMEM((N_FEATURES,), jnp.float32),
            pltpu.VMEM((BCH,), jnp.float32),
        ],
    )(_psqt_body)


def _psqt_body(icst_hbm, psqt_hbm, outp_hbm, idxt, psqt_v, pacc):
    wid = lax.axis_index("s") * NC + lax.axis_index("c")
    pltpu.sync_copy(psqt_hbm, psqt_v)

    def chunk_body(ci, carry):
        base = wid * BAGS_PER_W + ci * BCH
        pltpu.sync_copy(icst_hbm.at[:, pl.ds(base, BCH)], idxt)
        for i in range(BCH // L):
            pacc[pl.ds(i * L, L)] = jnp.zeros((L,), jnp.float32)

        def psum(j, c1):
            for i in range(BCH // L):
                s = pl.ds(i * L, L)
                pacc[s] += plsc.load_gather(psqt_v, [idxt[j, s]])
            return c1

        lax.fori_loop(0, K, psum, 0)
        pltpu.sync_copy(pacc, outp_hbm.at[pl.ds(base, BCH)])
        return carry

    lax.fori_loop(0, NCH, chunk_body, 0)


def _fc_body(w_ref, b_ref, wp_ref, bp_ref, stm_ref, bias_ref, fcw_ref,
             fcb_ref, out_ref):
    bias = bias_ref[...]
    wfts = w_ref[...] + bias
    bfts = b_ref[...] + bias
    s = stm_ref[...]
    x1 = (1.0 - s) * wfts + s * bfts
    x2 = (1.0 - s) * bfts + s * wfts
    fcw = fcw_ref[...]
    fca, fcbb = fcw[:, :DH], fcw[:, DH:]
    acc = jnp.sum(jnp.clip(x1, 0.0, 1.0) * fca, axis=1, keepdims=True)
    acc = acc + jnp.sum(jnp.clip(x2, 0.0, 1.0) * fcbb, axis=1, keepdims=True)
    out_ref[...] = acc + fcb_ref[...] + (wp_ref[...] - bp_ref[...]) * (0.5 - s)


def kernel(wft_ics, bft_ics, stm, ft_weight, ft_bias, fc_w, fc_b):
    ics = jnp.concatenate([wft_ics, bft_ics], axis=0)
    ics_flat = ics.reshape(-1)
    ics_t = ics.T
    table = ft_weight[:, :DH]
    psqt_col = ft_weight[:, DH]
    acc = _make_embed_bag()(ics_flat, table)
    psqt = _make_psqt()(ics_t, psqt_col)
    psqt2 = psqt.reshape(NBAGS, 1)

    bm = 512
    nb = BATCH // bm
    out = pl.pallas_call(
        _fc_body,
        grid=(nb,),
        in_specs=[
            pl.BlockSpec((bm, DH), lambda i: (i, 0)),
            pl.BlockSpec((bm, DH), lambda i: (i + nb, 0)),
            pl.BlockSpec((bm, 1), lambda i: (i, 0)),
            pl.BlockSpec((bm, 1), lambda i: (i + nb, 0)),
            pl.BlockSpec((bm, 1), lambda i: (i, 0)),
            pl.BlockSpec((1, DH), lambda i: (0, 0)),
            pl.BlockSpec((1, 512), lambda i: (0, 0)),
            pl.BlockSpec((1, 1), lambda i: (0, 0)),
        ],
        out_specs=pl.BlockSpec((bm, 1), lambda i: (i, 0)),
        out_shape=jax.ShapeDtypeStruct((BATCH, 1), jnp.float32),
    )(acc, acc, psqt2, psqt2, stm, ft_bias[:DH].reshape(1, DH), fc_w,
      fc_b.reshape(1, 1))
    return out


# speedup vs baseline: 27.3126x; 1.0013x over previous
import functools

import jax
import jax.numpy as jnp
from jax import lax
from jax.experimental import pallas as pl
from jax.experimental.pallas import tpu as pltpu
from jax.experimental.pallas import tpu_sc as plsc

N_FEATURES = 40960
D = 257
DH = 256
BATCH = 16384
K = 32
NBAGS = 2 * BATCH
NC, NS = 2, 16
NW = NC * NS
BAGS_PER_W = NBAGS // NW
BCH = 128
NCH = BAGS_PER_W // BCH
GROWS = 128
GBAGS = GROWS // K
NG = BCH // GBAGS
ACH = 32
L = 16


def _mesh():
    return plsc.VectorSubcoreMesh(
        core_axis_name="c", subcore_axis_name="s", num_cores=NC, num_subcores=NS
    )


@functools.cache
def _make_embed_bag():
    return functools.partial(
        pl.kernel,
        out_type=(
            jax.ShapeDtypeStruct((NBAGS, DH), jnp.float32),
            jax.ShapeDtypeStruct((NBAGS,), jnp.float32),
        ),
        mesh=_mesh(),
        compiler_params=pltpu.CompilerParams(needs_layout_passes=False),
        scratch_types=[
            pltpu.VMEM((BCH * K,), jnp.int32),
            pltpu.VMEM((K, BCH), jnp.int32),
            pltpu.VMEM((2, GROWS, DH), jnp.float32),
            pltpu.VMEM((ACH, DH), jnp.float32),
            pltpu.VMEM((N_FEATURES,), jnp.float32),
            pltpu.VMEM((BCH,), jnp.float32),
            pltpu.SemaphoreType.DMA,
            pltpu.SemaphoreType.DMA,
        ],
    )(_embed_bag_body)


def _embed_bag_body(icsf_hbm, icst_hbm, table_hbm, psqt_hbm,
                    out_hbm, outp_hbm,
                    idxf, idxt, rows, acc, psqt_v, pacc, sem0, sem1):
    wid = lax.axis_index("s") * NC + lax.axis_index("c")
    pltpu.sync_copy(psqt_hbm, psqt_v)
    sems = (sem0, sem1)

    def fire(g, buf):
        pltpu.async_copy(
            table_hbm.at[idxf.at[pl.ds(g * GROWS, GROWS)], pl.ds(0, DH)],
            rows.at[buf], sems[buf])

    def wait(buf):
        pltpu.make_async_copy(
            table_hbm.at[idxf.at[pl.ds(0, GROWS)], pl.ds(0, DH)],
            rows.at[buf], sems[buf]).wait()

    def reduce_buf(g, buf):
        rb = rows.at[buf]

        def bag_body(bag, c1):
            r0 = bag * K
            for cb in range(DH // L):
                s = pl.ds(cb * L, L)
                a0 = rb[r0, s] + rb[r0 + 4, s]
                a1 = rb[r0 + 1, s] + rb[r0 + 5, s]
                a2 = rb[r0 + 2, s] + rb[r0 + 6, s]
                a3 = rb[r0 + 3, s] + rb[r0 + 7, s]
                for r in range(8, K, 4):
                    a0 = a0 + rb[r0 + r, s]
                    a1 = a1 + rb[r0 + r + 1, s]
                    a2 = a2 + rb[r0 + r + 2, s]
                    a3 = a3 + rb[r0 + r + 3, s]
                acc[(g % (ACH // GBAGS)) * GBAGS + bag, s] = (a0 + a1) + (a2 + a3)
            return c1

        lax.fori_loop(0, GBAGS, bag_body, 0)

    def chunk_body(ci, carry):
        base = wid * BAGS_PER_W + ci * BCH
        pltpu.sync_copy(icsf_hbm.at[pl.ds(base * K, BCH * K)], idxf)
        pltpu.sync_copy(icst_hbm.at[:, pl.ds(base, BCH)], idxt)
        fire(0, 0)
        fire(1, 1)

        for i in range(BCH // L):
            pacc[pl.ds(i * L, L)] = jnp.zeros((L,), jnp.float32)

        def psum(j, c1):
            for i in range(BCH // L):
                s = pl.ds(i * L, L)
                pacc[s] += plsc.load_gather(psqt_v, [idxt[j, s]])
            return c1

        lax.fori_loop(0, K, psum, 0)

        def pipe_body(g2, c1):
            g = 2 * g2
            wait(0)
            reduce_buf(g, 0)

            @pl.when(g2 != NG // 2 - 1)
            def _():
                fire(g + 2, 0)

            wait(1)
            reduce_buf(g + 1, 1)

            @pl.when(g2 != NG // 2 - 1)
            def _():
                fire(g + 3, 1)

            @pl.when((g2 % (ACH // GBAGS // 2)) == ACH // GBAGS // 2 - 1)
            def _():
                blk = g2 // (ACH // GBAGS // 2)
                pltpu.sync_copy(acc, out_hbm.at[pl.ds(base + blk * ACH, ACH)])

            return c1

        lax.fori_loop(0, NG // 2, pipe_body, 0)
        pltpu.sync_copy(pacc, outp_hbm.at[pl.ds(base, BCH)])
        return carry

    lax.fori_loop(0, NCH, chunk_body, 0)


def _fc_body(w_ref, b_ref, wp_ref, bp_ref, stm_ref, bias_ref, fcw_ref,
             fcb_ref, out_ref):
    bias = bias_ref[...]
    wfts = w_ref[...] + bias
    bfts = b_ref[...] + bias
    s = stm_ref[...]
    x1 = (1.0 - s) * wfts + s * bfts
    x2 = (1.0 - s) * bfts + s * wfts
    fcw = fcw_ref[...]
    fca, fcbb = fcw[:, :DH], fcw[:, DH:]
    acc = jnp.sum(jnp.clip(x1, 0.0, 1.0) * fca, axis=1, keepdims=True)
    acc = acc + jnp.sum(jnp.clip(x2, 0.0, 1.0) * fcbb, axis=1, keepdims=True)
    out_ref[...] = acc + fcb_ref[...] + (wp_ref[...] - bp_ref[...]) * (0.5 - s)


def kernel(wft_ics, bft_ics, stm, ft_weight, ft_bias, fc_w, fc_b):
    ics = jnp.concatenate([wft_ics, bft_ics], axis=0)
    ics_flat = ics.reshape(-1)
    ics_t = ics.T
    psqt_col = ft_weight[:, DH]
    acc, psqt = _make_embed_bag()(ics_flat, ics_t, ft_weight, psqt_col)
    psqt2 = psqt.reshape(NBAGS, 1)

    bm = 512
    nb = BATCH // bm
    out = pl.pallas_call(
        _fc_body,
        grid=(nb,),
        in_specs=[
            pl.BlockSpec((bm, DH), lambda i: (i, 0)),
            pl.BlockSpec((bm, DH), lambda i: (i + nb, 0)),
            pl.BlockSpec((bm, 1), lambda i: (i, 0)),
            pl.BlockSpec((bm, 1), lambda i: (i + nb, 0)),
            pl.BlockSpec((bm, 1), lambda i: (i, 0)),
            pl.BlockSpec((1, DH), lambda i: (0, 0)),
            pl.BlockSpec((1, 512), lambda i: (0, 0)),
            pl.BlockSpec((1, 1), lambda i: (0, 0)),
        ],
        out_specs=pl.BlockSpec((bm, 1), lambda i: (i, 0)),
        out_shape=jax.ShapeDtypeStruct((BATCH, 1), jnp.float32),
    )(acc, acc, psqt2, psqt2, stm, ft_bias[:DH].reshape(1, DH), fc_w,
      fc_b.reshape(1, 1))
    return out
